# baseline (speedup 1.0000x reference)
import functools

import jax
import jax.numpy as jnp
from jax import lax
from jax.experimental import pallas as pl
from jax.experimental.pallas import tpu as pltpu
from jax.experimental.pallas import tpu_sc as plsc

_B = 16
_G = 76
_GG = _G * _G
_NA = 3
_NATTR = 85
_STRIDE = 8.0
_AW = (116.0, 156.0, 373.0)
_AH = (90.0, 198.0, 326.0)

_ROWS = 4
_P = _ROWS * _G
_NCH = _G // _ROWS
_NCHUNKS = _B * _NA * _NCH
_NW = 32


def _sigmoid(v):
    return 1.0 / (1.0 + jnp.exp(-v))


def _sc_yolo(x_hbm, out_hbm, in_v, out_v, sem):
    cid = lax.axis_index("c")
    sid = lax.axis_index("s")
    wid = sid * 2 + cid
    i16 = lax.broadcasted_iota(jnp.int32, (16,), 0)

    def chunk_body(i, carry):
        chunk = i * _NW + wid

        @pl.when(chunk < _NCHUNKS)
        def _do_chunk():
            b = (chunk * 1150) >> 16
            rem = chunk - b * 57
            a = (rem * 3450) >> 16
            t = rem - a * 19
            p0 = t * _P

            pltpu.async_copy(
                x_hbm.at[b, pl.ds(a * _NATTR, _NATTR), pl.ds(p0, _P)],
                in_v, sem).wait()

            aw = jnp.where(a == 0, _AW[0], jnp.where(a == 1, _AW[1], _AW[2]))
            ah = jnp.where(a == 0, _AH[0], jnp.where(a == 1, _AH[1], _AH[2]))

            for k in range(19):
                p = k * 16 + i16
                gyo = (p * 863) >> 16
                gxf = (p - gyo * _G).astype(jnp.float32)
                gyf = (t * _ROWS + gyo).astype(jnp.float32)
                v0 = in_v[0, pl.ds(k * 16, 16)]
                plsc.store_scatter(
                    out_v, [p, jnp.full((16,), 0, jnp.int32)],
                    (_sigmoid(v0) + gxf) * _STRIDE)
                v1 = in_v[1, pl.ds(k * 16, 16)]
                plsc.store_scatter(
                    out_v, [p, jnp.full((16,), 1, jnp.int32)],
                    (_sigmoid(v1) + gyf) * _STRIDE)
                v2 = in_v[2, pl.ds(k * 16, 16)]
                plsc.store_scatter(
                    out_v, [p, jnp.full((16,), 2, jnp.int32)],
                    jnp.minimum(jnp.exp(v2), 1000.0) * aw)
                v3 = in_v[3, pl.ds(k * 16, 16)]
                plsc.store_scatter(
                    out_v, [p, jnp.full((16,), 3, jnp.int32)],
                    jnp.minimum(jnp.exp(v3), 1000.0) * ah)

            def ch_body(c, inner):
                cvec = jnp.full((16,), c, jnp.int32)
                for k in range(19):
                    v = in_v[c, pl.ds(k * 16, 16)]
                    plsc.store_scatter(out_v, [k * 16 + i16, cvec],
                                       _sigmoid(v))
                return inner

            lax.fori_loop(4, _NATTR, ch_body, 0)

            pltpu.sync_copy(out_v, out_hbm.at[b, pl.ds(a * _GG + p0, _P)])

        return carry

    lax.fori_loop(0, (_NCHUNKS + _NW - 1) // _NW, chunk_body, 0)


def kernel(x, img_size):
    del img_size
    mesh = plsc.VectorSubcoreMesh(core_axis_name="c", subcore_axis_name="s")
    f = functools.partial(
        pl.kernel,
        mesh=mesh,
        out_type=jax.ShapeDtypeStruct((_B, _NA * _GG, _NATTR), jnp.float32),
        scratch_types=[
            pltpu.VMEM((_NATTR, _P), jnp.float32),
            pltpu.VMEM((_P, _NATTR), jnp.float32),
            pltpu.SemaphoreType.DMA,
        ],
        compiler_params=pltpu.CompilerParams(use_tc_tiling_on_sc=False, needs_layout_passes=False),
    )(_sc_yolo)
    return f(x.reshape(_B, _NA * _NATTR, _GG))


# speedup vs baseline: 1.4538x; 1.4538x over previous
import functools

import jax
import jax.numpy as jnp
from jax import lax
from jax.experimental import pallas as pl
from jax.experimental.pallas import tpu as pltpu
from jax.experimental.pallas import tpu_sc as plsc

_B = 16
_G = 76
_GG = _G * _G
_NA = 3
_NATTR = 85
_STRIDE = 8.0
_AW = (116.0, 156.0, 373.0)
_AH = (90.0, 198.0, 326.0)

_ROWS = 4
_P = _ROWS * _G
_NCH = _G // _ROWS
_NCHUNKS = _B * _NA * _NCH
_NW = 32


def _sigmoid(v):
    return 1.0 / (1.0 + jnp.exp(-v))


def _sc_yolo(x_hbm, out_hbm, in_v, out_v, sem):
    cid = lax.axis_index("c")
    sid = lax.axis_index("s")
    wid = sid * 2 + cid
    i16 = lax.broadcasted_iota(jnp.int32, (16,), 0)

    def chunk_body(i, carry):
        chunk = i * _NW + wid

        @pl.when(chunk < _NCHUNKS)
        def _do_chunk():
            b = (chunk * 1150) >> 16
            rem = chunk - b * 57
            a = (rem * 3450) >> 16
            t = rem - a * 19
            p0 = t * _P

            pltpu.async_copy(
                x_hbm.at[b, pl.ds(a * _NATTR, _NATTR), pl.ds(p0, _P)],
                in_v, sem).wait()

            aw = jnp.where(a == 0, _AW[0], jnp.where(a == 1, _AW[1], _AW[2]))
            ah = jnp.where(a == 0, _AH[0], jnp.where(a == 1, _AH[1], _AH[2]))

            for k in range(19):
                p = k * 16 + i16
                gyo = (p * 863) >> 16
                gxf = (p - gyo * _G).astype(jnp.float32)
                gyf = (t * _ROWS + gyo).astype(jnp.float32)
                v0 = in_v[0, pl.ds(k * 16, 16)]
                plsc.store_scatter(
                    out_v, [p, jnp.full((16,), 0, jnp.int32)],
                    (_sigmoid(v0) + gxf) * _STRIDE)
                v1 = in_v[1, pl.ds(k * 16, 16)]
                plsc.store_scatter(
                    out_v, [p, jnp.full((16,), 1, jnp.int32)],
                    (_sigmoid(v1) + gyf) * _STRIDE)
                v2 = in_v[2, pl.ds(k * 16, 16)]
                plsc.store_scatter(
                    out_v, [p, jnp.full((16,), 2, jnp.int32)],
                    jnp.minimum(jnp.exp(v2), 1000.0) * aw)
                v3 = in_v[3, pl.ds(k * 16, 16)]
                plsc.store_scatter(
                    out_v, [p, jnp.full((16,), 3, jnp.int32)],
                    jnp.minimum(jnp.exp(v3), 1000.0) * ah)

            @plsc.parallel_loop(4, _NATTR, unroll=3)
            def ch_body(c):
                cvec = jnp.full((16,), c, jnp.int32)
                for k in range(19):
                    v = in_v[c, pl.ds(k * 16, 16)]
                    plsc.store_scatter(out_v, [k * 16 + i16, cvec],
                                       _sigmoid(v))

            pltpu.sync_copy(out_v, out_hbm.at[b, pl.ds(a * _GG + p0, _P)])

        return carry

    lax.fori_loop(0, (_NCHUNKS + _NW - 1) // _NW, chunk_body, 0)


def kernel(x, img_size):
    del img_size
    mesh = plsc.VectorSubcoreMesh(core_axis_name="c", subcore_axis_name="s")
    f = functools.partial(
        pl.kernel,
        mesh=mesh,
        out_type=jax.ShapeDtypeStruct((_B, _NA * _GG, _NATTR), jnp.float32),
        scratch_types=[
            pltpu.VMEM((_NATTR, _P), jnp.float32),
            pltpu.VMEM((_P, _NATTR), jnp.float32),
            pltpu.SemaphoreType.DMA,
        ],
        compiler_params=pltpu.CompilerParams(use_tc_tiling_on_sc=False, needs_layout_passes=False),
    )(_sc_yolo)
    return f(x.reshape(_B, _NA * _NATTR, _GG))


# speedup vs baseline: 2.9557x; 2.0331x over previous
import functools

import jax
import jax.numpy as jnp
from jax import lax
from jax.experimental import pallas as pl
from jax.experimental.pallas import tpu as pltpu
from jax.experimental.pallas import tpu_sc as plsc

_B = 16
_G = 76
_GG = _G * _G
_NA = 3
_NATTR = 85
_STRIDE = 8.0
_AW = (116.0, 156.0, 373.0)
_AH = (90.0, 198.0, 326.0)

_TR = 8
_HP = 304
_NT = 10
_NCHUNKS = _B * _NA * _NT
_NW = 32


def _sigmoid(v):
    return 1.0 / (1.0 + jnp.exp(-v))


def _sc_yolo(x_hbm, out_hbm, in_v, out_v, sem):
    cid = lax.axis_index("c")
    sid = lax.axis_index("s")
    wid = sid * 2 + cid
    i16 = lax.broadcasted_iota(jnp.int32, (16,), 0)

    def chunk_body(i, carry):
        chunk = i * _NW + wid
        b = (chunk * 2185) >> 16
        rem = chunk - b * 30
        a = (rem * 6554) >> 16
        t = rem - a * 10
        gy0 = t * _TR

        pltpu.async_copy(
            x_hbm.at[b, pl.ds(a * _NATTR, _NATTR), pl.ds(gy0, _TR)],
            in_v, sem).wait()

        aw = jnp.where(a == 0, _AW[0], jnp.where(a == 1, _AW[1], _AW[2]))
        ah = jnp.where(a == 0, _AH[0], jnp.where(a == 1, _AH[1], _AH[2]))

        def do_half(h):
            for v in range(19):
                p = v * 16 + i16
                gyo = (p * 863) >> 16
                gx = p - gyo * _G
                j = gyo + (h * 4)
                gxf = gx.astype(jnp.float32)
                gyf = (gy0 + j).astype(jnp.float32)

                def ld(c_vec):
                    return plsc.load_gather(in_v, [c_vec, j, gx])

                c0 = jnp.full((16,), 0, jnp.int32)
                plsc.store_scatter(out_v, [p, c0],
                                   (_sigmoid(ld(c0)) + gxf) * _STRIDE)
                c1 = jnp.full((16,), 1, jnp.int32)
                plsc.store_scatter(out_v, [p, c1],
                                   (_sigmoid(ld(c1)) + gyf) * _STRIDE)
                c2 = jnp.full((16,), 2, jnp.int32)
                plsc.store_scatter(
                    out_v, [p, c2],
                    jnp.minimum(jnp.exp(ld(c2)), 1000.0) * aw)
                c3 = jnp.full((16,), 3, jnp.int32)
                plsc.store_scatter(
                    out_v, [p, c3],
                    jnp.minimum(jnp.exp(ld(c3)), 1000.0) * ah)

                @plsc.parallel_loop(4, _NATTR, unroll=3)
                def ch_body(c):
                    cvec = jnp.full((16,), c, jnp.int32)
                    plsc.store_scatter(out_v, [p, cvec],
                                       _sigmoid(ld(cvec)))

            row0 = a * _GG + gy0 * _G + h * _HP
            pltpu.sync_copy(out_v, out_hbm.at[b, pl.ds(row0, _HP)])

        do_half(0)

        @pl.when(t < _NT - 1)
        def _half1():
            do_half(1)

        return carry

    lax.fori_loop(0, _NCHUNKS // _NW, chunk_body, 0)


def kernel(x, img_size):
    del img_size
    mesh = plsc.VectorSubcoreMesh(core_axis_name="c", subcore_axis_name="s")
    f = functools.partial(
        pl.kernel,
        mesh=mesh,
        out_type=jax.ShapeDtypeStruct((_B, _NA * _GG, _NATTR), jnp.float32),
        scratch_types=[
            pltpu.VMEM((_NATTR, _TR, _G), jnp.float32),
            pltpu.VMEM((_HP, _NATTR), jnp.float32),
            pltpu.SemaphoreType.DMA,
        ],
        compiler_params=pltpu.CompilerParams(
            use_tc_tiling_on_sc=True, needs_layout_passes=False),
    )(_sc_yolo)
    return f(x)


# speedup vs baseline: 3.0241x; 1.0231x over previous
import functools

import jax
import jax.numpy as jnp
from jax import lax
from jax.experimental import pallas as pl
from jax.experimental.pallas import tpu as pltpu
from jax.experimental.pallas import tpu_sc as plsc

_B = 16
_G = 76
_GG = _G * _G
_NA = 3
_NATTR = 85
_STRIDE = 8.0
_AW = (116.0, 156.0, 373.0)
_AH = (90.0, 198.0, 326.0)

_TR = 8
_HP = 304
_NT = 10
_NCHUNKS = _B * _NA * _NT
_NW = 32


def _sigmoid(v):
    return 1.0 / (1.0 + jnp.exp(-v))


def _sc_yolo(x_hbm, out_hbm, in_v, out_v, sem):
    cid = lax.axis_index("c")
    sid = lax.axis_index("s")
    wid = sid * 2 + cid
    i16 = lax.broadcasted_iota(jnp.int32, (16,), 0)

    def chunk_body(i, carry):
        chunk = i * _NW + wid
        b = (chunk * 2185) >> 16
        rem = chunk - b * 30
        a = (rem * 6554) >> 16
        t = rem - a * 10
        gy0 = t * _TR

        pltpu.async_copy(
            x_hbm.at[b, pl.ds(a * _NATTR, _NATTR), pl.ds(gy0, _TR)],
            in_v, sem).wait()

        aw = jnp.where(a == 0, _AW[0], jnp.where(a == 1, _AW[1], _AW[2]))
        ah = jnp.where(a == 0, _AH[0], jnp.where(a == 1, _AH[1], _AH[2]))

        def do_half(h):
            for v in range(19):
                p = v * 16 + i16
                gyo = (p * 863) >> 16
                gx = p - gyo * _G
                j = gyo + (h * 4)
                gxf = gx.astype(jnp.float32)
                gyf = (gy0 + j).astype(jnp.float32)

                def ld(c_vec):
                    return plsc.load_gather(in_v, [c_vec, j, gx])

                c0 = jnp.full((16,), 0, jnp.int32)
                plsc.store_scatter(out_v, [p, c0],
                                   (_sigmoid(ld(c0)) + gxf) * _STRIDE)
                c1 = jnp.full((16,), 1, jnp.int32)
                plsc.store_scatter(out_v, [p, c1],
                                   (_sigmoid(ld(c1)) + gyf) * _STRIDE)
                c2 = jnp.full((16,), 2, jnp.int32)
                plsc.store_scatter(
                    out_v, [p, c2],
                    jnp.minimum(jnp.exp(ld(c2)), 1000.0) * aw)
                c3 = jnp.full((16,), 3, jnp.int32)
                plsc.store_scatter(
                    out_v, [p, c3],
                    jnp.minimum(jnp.exp(ld(c3)), 1000.0) * ah)

            @plsc.parallel_loop(4, _NATTR, unroll=2)
            def ch_body(c):
                cvec = jnp.full((16,), c, jnp.int32)
                for v in range(19):
                    p = v * 16 + i16
                    gyo = (p * 863) >> 16
                    gx = p - gyo * _G
                    j = gyo + (h * 4)
                    plsc.store_scatter(
                        out_v, [p, cvec],
                        _sigmoid(plsc.load_gather(in_v, [cvec, j, gx])))

            row0 = a * _GG + gy0 * _G + h * _HP
            pltpu.sync_copy(out_v, out_hbm.at[b, pl.ds(row0, _HP)])

        do_half(0)

        @pl.when(t < _NT - 1)
        def _half1():
            do_half(1)

        return carry

    lax.fori_loop(0, _NCHUNKS // _NW, chunk_body, 0)


def kernel(x, img_size):
    del img_size
    mesh = plsc.VectorSubcoreMesh(core_axis_name="c", subcore_axis_name="s")
    f = functools.partial(
        pl.kernel,
        mesh=mesh,
        out_type=jax.ShapeDtypeStruct((_B, _NA * _GG, _NATTR), jnp.float32),
        scratch_types=[
            pltpu.VMEM((_NATTR, _TR, _G), jnp.float32),
            pltpu.VMEM((_HP, _NATTR), jnp.float32),
            pltpu.SemaphoreType.DMA,
        ],
        compiler_params=pltpu.CompilerParams(
            use_tc_tiling_on_sc=True, needs_layout_passes=False),
    )(_sc_yolo)
    return f(x)
